# Initial kernel scaffold; baseline (speedup 1.0000x reference)
import jax
import jax.numpy as jnp
from jax import lax
from jax.experimental import pallas as pl
from jax.experimental.pallas import tpu as pltpu
from jax.experimental.pallas import tpu_sc as plsc

NN = 50000
EE = 800000
EPAD = 819200
NPAD = NN + 128
NGROUPS = 64
BLK = 1000
GRID = NN // BLK
ROWS_PER_TEC = NN // 16


_SC_MESH = dict(core_axis_name="c", subcore_axis_name="s")


def _deg_body(dst2_hbm, out_hbm, ones_v, didx_v, acc_s):
    c = lax.axis_index("c")
    s = lax.axis_index("s")

    def fill(i, carry):
        ones_v[i, pl.ds(0, 16)] = jnp.full((16,), 1.0, jnp.float32)
        return carry

    lax.fori_loop(0, 625, fill, 0, unroll=4)
    r0 = s * ROWS_PER_TEC
    for t in range(5):
        pltpu.sync_copy(ones_v, acc_s.at[pl.ds(r0 + t * 625, 625)])
    plsc.subcore_barrier()

    def blk(b, carry):
        er0 = c * 3200 + s * 200 + b * 8
        pltpu.sync_copy(dst2_hbm.at[pl.ds(er0, 8)], didx_v)
        for j in range(8):
            pltpu.sync_copy(ones_v.at[pl.ds(0, 128)], acc_s.at[didx_v.at[j]],
                            add=True)
        return carry

    lax.fori_loop(0, 25, blk, 0)
    plsc.subcore_barrier()
    pltpu.sync_copy(acc_s.at[pl.ds(r0, ROWS_PER_TEC)],
                    out_hbm.at[pl.ds(c * NN + r0, ROWS_PER_TEC)])


_deg_call = pl.kernel(
    _deg_body,
    out_type=jax.ShapeDtypeStruct((2 * NN, 16), jnp.float32),
    mesh=plsc.VectorSubcoreMesh(**_SC_MESH),
    scratch_types=[
        pltpu.VMEM((625, 16), jnp.float32),
        pltpu.VMEM((8, 128), jnp.int32),
        pltpu.VMEM_SHARED((NPAD, 16), jnp.float32),
    ],
)


def _make_spmm(S):
    S_pc = S // 2

    def body(u_hbm, src2_hbm, dst2_hbm, out_hbm, sidx_v, didx_v, gidx_v,
             rows_v, sems, acc_s):
        c = lax.axis_index("c")
        s = lax.axis_index("s")

        def slice_body(sl_local, carry):
            sl = c * S_pc + sl_local
            base_row = sl * NN
            r0 = s * ROWS_PER_TEC
            pltpu.sync_copy(u_hbm.at[pl.ds(base_row + r0, ROWS_PER_TEC)],
                            acc_s.at[pl.ds(r0, ROWS_PER_TEC)])
            plsc.subcore_barrier()

            def blk(b, bcarry):
                er0 = s * 400 + b * 16
                pltpu.sync_copy(src2_hbm.at[pl.ds(er0, 16)], sidx_v)
                pltpu.sync_copy(dst2_hbm.at[pl.ds(er0, 16)], didx_v)

                def addo(i, icarry):
                    j = i // 8
                    k = (i % 8) * 16
                    gidx_v[j, pl.ds(k, 16)] = (
                        sidx_v[j, pl.ds(k, 16)] + base_row)
                    return icarry

                lax.fori_loop(0, 128, addo, 0, unroll=4)
                cps = [
                    pltpu.async_copy(u_hbm.at[gidx_v.at[j]], rows_v.at[j],
                                     sems.at[j]) for j in range(16)
                ]
                for j in range(16):
                    cps[j].wait()
                    pltpu.sync_copy(rows_v.at[j], acc_s.at[didx_v.at[j]],
                                    add=True)
                return bcarry

            lax.fori_loop(0, 25, blk, 0)
            plsc.subcore_barrier()
            pltpu.sync_copy(acc_s.at[pl.ds(r0, ROWS_PER_TEC)],
                            out_hbm.at[pl.ds(base_row + r0, ROWS_PER_TEC)])
            plsc.subcore_barrier()
            return carry

        lax.fori_loop(0, S_pc, slice_body, 0)

    return pl.kernel(
        body,
        out_type=jax.ShapeDtypeStruct((S * NN, 32), jnp.float32),
        mesh=plsc.VectorSubcoreMesh(**_SC_MESH),
        scratch_types=[
            pltpu.VMEM((16, 128), jnp.int32),
            pltpu.VMEM((16, 128), jnp.int32),
            pltpu.VMEM((16, 128), jnp.int32),
            pltpu.VMEM((16, 128, 32), jnp.float32),
            pltpu.SemaphoreType.DMA((16,)),
            pltpu.VMEM_SHARED((NPAD, 32), jnp.float32),
        ],
    )


_spmm2 = _make_spmm(2)
_spmm4 = _make_spmm(4)

NP2 = 50176
RPW = 1568
RBLK = 224
NBLKP = RPW // RBLK


def _pool_body(h_hbm, b_hbm, out_hbm, rows_v, bid_v, pacc_v):
    c = lax.axis_index("c")
    s = lax.axis_index("s")
    w = s * 2 + c

    def initp(i, carry):
        for j in range(4):
            pacc_v[i, pl.ds(j * 16, 16)] = jnp.full((16,), -jnp.inf,
                                                    jnp.float32)
        return carry

    lax.fori_loop(0, NGROUPS, initp, 0, unroll=4)
    pltpu.sync_copy(b_hbm.at[pl.ds(w * RPW, RPW)], bid_v)

    def blk(bb, carry):
        pltpu.sync_copy(h_hbm.at[pl.ds(w * RPW + bb * RBLK, RBLK)], rows_v)

        def row(r, rcarry):
            gid = bid_v[bb * RBLK + r]
            for j in range(4):
                sl_ = pl.ds(j * 16, 16)
                pacc_v[gid, sl_] = jnp.maximum(pacc_v[gid, sl_],
                                               rows_v[r, sl_])
            return rcarry

        lax.fori_loop(0, RBLK, row, 0)
        return carry

    lax.fori_loop(0, NBLKP, blk, 0)
    pltpu.sync_copy(pacc_v, out_hbm.at[w])


_pool_call = pl.kernel(
    _pool_body,
    out_type=jax.ShapeDtypeStruct((32, NGROUPS, 64), jnp.float32),
    mesh=plsc.VectorSubcoreMesh(**_SC_MESH),
    scratch_types=[
        pltpu.VMEM((RBLK, 64), jnp.float32),
        pltpu.VMEM((RPW,), jnp.int32),
        pltpu.VMEM((NGROUPS, 64), jnp.float32),
    ],
)


def _gelu_f(x):
    return x * 0.5 * (1.0 + lax.erf(x * 0.7071067811865476))


def _dinv_body(dp_ref, o_ref):
    d = dp_ref[0, :, 0:1] + dp_ref[1, :, 0:1] - 1.0
    o_ref[...] = lax.rsqrt(d)


def _dinv_call(deg_parts):
    return pl.pallas_call(
        _dinv_body,
        grid=(GRID,),
        in_specs=[pl.BlockSpec((2, BLK, 16), lambda i: (0, i, 0))],
        out_specs=pl.BlockSpec((BLK, 1), lambda i: (i, 0)),
        out_shape=jax.ShapeDtypeStruct((NN, 1), jnp.float32),
    )(deg_parts)


def _pre1_body(x_ref, w_ref, dv_ref, o_ref):
    u = jnp.dot(x_ref[...], w_ref[...],
                preferred_element_type=jnp.float32) * dv_ref[...]
    for i in range(2):
        o_ref[i] = u[:, 32 * i:32 * (i + 1)]


def _pre1_call(x, W1, dinv):
    return pl.pallas_call(
        _pre1_body,
        grid=(GRID,),
        in_specs=[
            pl.BlockSpec((BLK, 2), lambda i: (i, 0)),
            pl.BlockSpec((2, 64), lambda i: (0, 0)),
            pl.BlockSpec((BLK, 1), lambda i: (i, 0)),
        ],
        out_specs=pl.BlockSpec((2, BLK, 32), lambda i: (0, i, 0)),
        out_shape=jax.ShapeDtypeStruct((2, NN, 32), jnp.float32),
    )(x, W1, dinv)


def _make_post_pre(S_in, S_out, d_in, d_out):
    def body(a_ref, b_ref, dv_ref, w_ref, o_ref):
        acc = jnp.concatenate([a_ref[i] for i in range(S_in)], axis=1)
        dv = dv_ref[...]
        h = _gelu_f(dv * acc + b_ref[...])
        u = jnp.dot(h, w_ref[...], preferred_element_type=jnp.float32) * dv
        for i in range(S_out):
            o_ref[i] = u[:, 32 * i:32 * (i + 1)]

    def call(a, b, dinv, W):
        return pl.pallas_call(
            body,
            grid=(GRID,),
            in_specs=[
                pl.BlockSpec((S_in, BLK, 32), lambda i: (0, i, 0)),
                pl.BlockSpec((1, d_in), lambda i: (0, 0)),
                pl.BlockSpec((BLK, 1), lambda i: (i, 0)),
                pl.BlockSpec((d_in, d_out), lambda i: (0, 0)),
            ],
            out_specs=pl.BlockSpec((S_out, BLK, 32), lambda i: (0, i, 0)),
            out_shape=jax.ShapeDtypeStruct((S_out, NN, 32), jnp.float32),
        )(a, b.reshape(1, d_in), dinv, W)

    return call


def _make_bnstat(S_in, d):
    def body(a_ref, b_ref, dv_ref, z_ref, s_ref, q_ref):
        acc = jnp.concatenate([a_ref[i] for i in range(S_in)], axis=1)
        z = dv_ref[...] * acc + b_ref[...]
        z_ref[...] = z
        s_ref[...] = jnp.sum(z, axis=0, keepdims=True)
        q_ref[...] = jnp.sum(z * z, axis=0, keepdims=True)

    def call(a, b, dinv):
        return pl.pallas_call(
            body,
            grid=(GRID,),
            in_specs=[
                pl.BlockSpec((S_in, BLK, 32), lambda i: (0, i, 0)),
                pl.BlockSpec((1, d), lambda i: (0, 0)),
                pl.BlockSpec((BLK, 1), lambda i: (i, 0)),
            ],
            out_specs=[
                pl.BlockSpec((BLK, d), lambda i: (i, 0)),
                pl.BlockSpec((1, d), lambda i: (i, 0)),
                pl.BlockSpec((1, d), lambda i: (i, 0)),
            ],
            out_shape=[
                jax.ShapeDtypeStruct((NN, d), jnp.float32),
                jax.ShapeDtypeStruct((GRID, d), jnp.float32),
                jax.ShapeDtypeStruct((GRID, d), jnp.float32),
            ],
        )(a, b.reshape(1, d), dinv)

    return call


def _make_bnapply(d, S_out, d_out, with_pre):
    def body(z_ref, s_ref, q_ref, g_ref, be_ref, *rest):
        m = jnp.sum(s_ref[...], axis=0, keepdims=True) * (1.0 / NN)
        v = jnp.sum(q_ref[...], axis=0, keepdims=True) * (1.0 / NN) - m * m
        inv = lax.rsqrt(v + 1e-5)
        h = _gelu_f((z_ref[...] - m) * inv * g_ref[...] + be_ref[...])
        if with_pre:
            w_ref, dv_ref, o_ref = rest
            u = jnp.dot(h, w_ref[...],
                        preferred_element_type=jnp.float32) * dv_ref[...]
            for i in range(S_out):
                o_ref[i] = u[:, 32 * i:32 * (i + 1)]
        else:
            (o_ref,) = rest
            o_ref[...] = h

    def call(z, sums, sq, g, be, W=None, dinv=None):
        in_specs = [
            pl.BlockSpec((BLK, d), lambda i: (i, 0)),
            pl.BlockSpec((GRID, d), lambda i: (0, 0)),
            pl.BlockSpec((GRID, d), lambda i: (0, 0)),
            pl.BlockSpec((1, d), lambda i: (0, 0)),
            pl.BlockSpec((1, d), lambda i: (0, 0)),
        ]
        args = [z, sums, sq, g.reshape(1, d), be.reshape(1, d)]
        if with_pre:
            in_specs += [
                pl.BlockSpec((d, d_out), lambda i: (0, 0)),
                pl.BlockSpec((BLK, 1), lambda i: (i, 0)),
            ]
            args += [W, dinv]
            out_specs = pl.BlockSpec((S_out, BLK, 32), lambda i: (0, i, 0))
            out_shape = jax.ShapeDtypeStruct((S_out, NN, 32), jnp.float32)
        else:
            out_specs = pl.BlockSpec((BLK, d), lambda i: (i, 0))
            out_shape = jax.ShapeDtypeStruct((NN, d), jnp.float32)
        return pl.pallas_call(
            body,
            grid=(GRID,),
            in_specs=in_specs,
            out_specs=out_specs,
            out_shape=out_shape,
        )(*args)

    return call


def _head_body(p_ref, w1_ref, b1_ref, w2_ref, b2_ref, o_ref):
    pooled = jnp.max(p_ref[...], axis=0)
    t = jnp.dot(pooled, w1_ref[...],
                preferred_element_type=jnp.float32) + b1_ref[...]
    o_ref[...] = jnp.dot(t, w2_ref[...],
                         preferred_element_type=jnp.float32) + b2_ref[...]


def _head_call(parts, lin1_W, lin1_b, lin_W, lin_b):
    return pl.pallas_call(
        _head_body,
        out_shape=jax.ShapeDtypeStruct((NGROUPS, 2), jnp.float32),
    )(parts, lin1_W, lin1_b.reshape(1, 10), lin_W, lin_b.reshape(1, 2))


_bnstat64 = _make_bnstat(2, 64)
_bnstat128 = _make_bnstat(4, 128)
_bnapply64_pre = _make_bnapply(64, 2, 64, True)
_bnapply128_pre = _make_bnapply(128, 4, 128, True)
_bnapply64_out = _make_bnapply(64, 0, 0, False)
_post64_128 = _make_post_pre(2, 4, 64, 128)
_post128_64 = _make_post_pre(4, 2, 128, 64)


def kernel(x, ei, batch, W1, b1, W2, b2, W3, b3, W4, b4, W5, b5, g1, be1, g2,
           be2, g3, be3, lin1_W, lin1_b, lin_W, lin_b):
    f32 = jnp.float32
    src = ei[0]
    dst = ei[1]
    pad_idx = jnp.arange(EPAD - EE, dtype=jnp.int32) % 128
    src_p = jnp.concatenate([src, pad_idx]).reshape(EPAD // 128, 128)
    dst_p = jnp.concatenate([dst, pad_idx + NN]).reshape(EPAD // 128, 128)

    deg_parts = _deg_call(dst_p).reshape(2, NN, 16)
    dinv = _dinv_call(deg_parts)

    u1 = _pre1_call(x.astype(f32), W1, dinv)
    a1 = _spmm2(u1.reshape(2 * NN, 32), src_p, dst_p).reshape(2, NN, 32)
    z1, s1, q1 = _bnstat64(a1, b1, dinv)
    u2 = _bnapply64_pre(z1, s1, q1, g1, be1, W2, dinv)
    a2 = _spmm2(u2.reshape(2 * NN, 32), src_p, dst_p).reshape(2, NN, 32)
    u3 = _post64_128(a2, b2, dinv, W3)
    a3 = _spmm4(u3.reshape(4 * NN, 32), src_p, dst_p).reshape(4, NN, 32)
    z3, s3, q3 = _bnstat128(a3, b3, dinv)
    u4 = _bnapply128_pre(z3, s3, q3, g2, be2, W4, dinv)
    a4 = _spmm4(u4.reshape(4 * NN, 32), src_p, dst_p).reshape(4, NN, 32)
    u5 = _post128_64(a4, b4, dinv, W5)
    a5 = _spmm2(u5.reshape(2 * NN, 32), src_p, dst_p).reshape(2, NN, 32)
    z5, s5, q5 = _bnstat64(a5, b5, dinv)
    h5 = _bnapply64_out(z5, s5, q5, g3, be3)

    h5p = jnp.concatenate(
        [h5, jnp.full((NP2 - NN, 64), -jnp.inf, f32)], axis=0)
    bp = jnp.concatenate(
        [batch, jnp.full((NP2 - NN,), NGROUPS - 1, jnp.int32)])
    parts = _pool_call(h5p, bp)
    return _head_call(parts, lin1_W, lin1_b, lin_W, lin_b)


# speedup vs baseline: 13.3276x; 13.3276x over previous
import jax
import jax.numpy as jnp
from jax import lax
from jax.experimental import pallas as pl
from jax.experimental.pallas import tpu as pltpu
from jax.experimental.pallas import tpu_sc as plsc

NN = 50000
EE = 800000
EPAD = 819200
NPAD = NN + 128
NGROUPS = 64
BLK = 1000
GRID = NN // BLK
RPT = 3128
RPT_LAST = NN - 15 * RPT


def _row_split_copy(s, src_fn, dst_fn):
    @pl.when(s < 15)
    def _():
        pltpu.sync_copy(src_fn(RPT), dst_fn(RPT))

    @pl.when(s == 15)
    def _():
        pltpu.sync_copy(src_fn(RPT_LAST), dst_fn(RPT_LAST))


_SC_MESH = dict(core_axis_name="c", subcore_axis_name="s")


def _deg_body(dst2_hbm, out_hbm, ones_v, didx_v, acc_s):
    c = lax.axis_index("c")
    s = lax.axis_index("s")

    def fill(i, carry):
        ones_v[i, pl.ds(0, 16)] = jnp.full((16,), 1.0, jnp.float32)
        return carry

    lax.fori_loop(0, RPT, fill, 0, unroll=4)
    r0 = s * RPT
    _row_split_copy(s, lambda n: ones_v.at[pl.ds(0, n)],
                    lambda n: acc_s.at[pl.ds(r0, n)])
    plsc.subcore_barrier()

    def blk(b, carry):
        er0 = c * 3200 + s * 200 + b * 8
        pltpu.sync_copy(dst2_hbm.at[pl.ds(er0, 8)], didx_v)
        for j in range(8):
            pltpu.sync_copy(ones_v.at[pl.ds(0, 128)], acc_s.at[didx_v.at[j]],
                            add=True)
        return carry

    lax.fori_loop(0, 25, blk, 0)
    plsc.subcore_barrier()
    _row_split_copy(s, lambda n: acc_s.at[pl.ds(r0, n)],
                    lambda n: out_hbm.at[pl.ds(c * NN + r0, n)])


_deg_call = pl.kernel(
    _deg_body,
    out_type=jax.ShapeDtypeStruct((2 * NN, 16), jnp.float32),
    mesh=plsc.VectorSubcoreMesh(**_SC_MESH),
    compiler_params=pltpu.CompilerParams(use_tc_tiling_on_sc=False),
    scratch_types=[
        pltpu.VMEM((RPT, 16), jnp.float32),
        pltpu.VMEM((8, 128), jnp.int32),
        pltpu.VMEM_SHARED((NPAD, 16), jnp.float32),
    ],
)


def _make_spmm(S):
    S_pc = S // 2

    def body(u_hbm, src2_hbm, dst2_hbm, out_hbm, sidx_v, didx_v, gidx_v,
             rows_v, sems, acc_s):
        c = lax.axis_index("c")
        s = lax.axis_index("s")

        def slice_body(sl_local, carry):
            sl = c * S_pc + sl_local
            base_row = sl * NN
            r0 = s * RPT
            _row_split_copy(s, lambda n: u_hbm.at[pl.ds(base_row + r0, n)],
                            lambda n: acc_s.at[pl.ds(r0, n)])
            plsc.subcore_barrier()

            def blk(b, bcarry):
                er0 = s * 400 + b * 16
                pltpu.sync_copy(src2_hbm.at[pl.ds(er0, 16)], sidx_v)
                pltpu.sync_copy(dst2_hbm.at[pl.ds(er0, 16)], didx_v)

                def addo(i, icarry):
                    j = i // 8
                    k = (i % 8) * 16
                    gidx_v[j, pl.ds(k, 16)] = (
                        sidx_v[j, pl.ds(k, 16)] + base_row)
                    return icarry

                lax.fori_loop(0, 128, addo, 0, unroll=4)
                for g in range(4):
                    cps = [
                        pltpu.async_copy(u_hbm.at[gidx_v.at[4 * g + j]],
                                         rows_v.at[j], sems.at[j])
                        for j in range(4)
                    ]
                    for j in range(4):
                        cps[j].wait()
                        pltpu.sync_copy(rows_v.at[j],
                                        acc_s.at[didx_v.at[4 * g + j]],
                                        add=True)
                return bcarry

            lax.fori_loop(0, 25, blk, 0)
            plsc.subcore_barrier()
            _row_split_copy(s, lambda n: acc_s.at[pl.ds(r0, n)],
                            lambda n: out_hbm.at[pl.ds(base_row + r0, n)])
            plsc.subcore_barrier()
            return carry

        lax.fori_loop(0, S_pc, slice_body, 0)

    return pl.kernel(
        body,
        out_type=jax.ShapeDtypeStruct((S * NN, 32), jnp.float32),
        mesh=plsc.VectorSubcoreMesh(**_SC_MESH),
        compiler_params=pltpu.CompilerParams(use_tc_tiling_on_sc=False),
        scratch_types=[
            pltpu.VMEM((16, 128), jnp.int32),
            pltpu.VMEM((16, 128), jnp.int32),
            pltpu.VMEM((16, 128), jnp.int32),
            pltpu.VMEM((4, 128, 32), jnp.float32),
            pltpu.SemaphoreType.DMA((4,)),
            pltpu.VMEM_SHARED((NPAD, 32), jnp.float32),
        ],
    )


_spmm2 = _make_spmm(2)
_spmm4 = _make_spmm(4)

NP2 = 50176
RPW = 1568
RBLK = 224
NBLKP = RPW // RBLK


def _pool_body(h_hbm, b_hbm, out_hbm, rows_v, bid_v, pacc_v):
    c = lax.axis_index("c")
    s = lax.axis_index("s")
    w = s * 2 + c

    def initp(i, carry):
        for j in range(4):
            pacc_v[i, pl.ds(j * 16, 16)] = jnp.full((16,), -jnp.inf,
                                                    jnp.float32)
        return carry

    lax.fori_loop(0, NGROUPS, initp, 0, unroll=4)
    pltpu.sync_copy(b_hbm.at[pl.ds(w * RPW, RPW)], bid_v.at[pl.ds(0, RPW)])

    def blk(bb, carry):
        pltpu.sync_copy(h_hbm.at[pl.ds(w * RPW + bb * RBLK, RBLK)], rows_v)

        def row(r, rcarry):
            gid = bid_v[pl.ds(bb * RBLK + r, 16)][0]
            for j in range(4):
                sl_ = pl.ds(j * 16, 16)
                pacc_v[gid, sl_] = jnp.maximum(pacc_v[gid, sl_],
                                               rows_v[r, sl_])
            return rcarry

        lax.fori_loop(0, RBLK, row, 0)
        return carry

    lax.fori_loop(0, NBLKP, blk, 0)
    pltpu.sync_copy(pacc_v, out_hbm.at[w])


_pool_call = pl.kernel(
    _pool_body,
    out_type=jax.ShapeDtypeStruct((32, NGROUPS, 64), jnp.float32),
    mesh=plsc.VectorSubcoreMesh(**_SC_MESH),
    compiler_params=pltpu.CompilerParams(use_tc_tiling_on_sc=False),
    scratch_types=[
        pltpu.VMEM((RBLK, 64), jnp.float32),
        pltpu.VMEM((RPW + 16,), jnp.int32),
        pltpu.VMEM((NGROUPS, 64), jnp.float32),
    ],
)


def _gelu_f(x):
    return x * 0.5 * (1.0 + lax.erf(x * 0.7071067811865476))


def _dinv_body(dp_ref, o_ref):
    d = dp_ref[0, :, 0:1] + dp_ref[1, :, 0:1] - 1.0
    o_ref[...] = lax.rsqrt(d)


def _dinv_call(deg_parts):
    return pl.pallas_call(
        _dinv_body,
        grid=(GRID,),
        in_specs=[pl.BlockSpec((2, BLK, 16), lambda i: (0, i, 0))],
        out_specs=pl.BlockSpec((BLK, 1), lambda i: (i, 0)),
        out_shape=jax.ShapeDtypeStruct((NN, 1), jnp.float32),
    )(deg_parts)


def _pre1_body(x_ref, w_ref, dv_ref, o_ref):
    u = jnp.dot(x_ref[...], w_ref[...],
                preferred_element_type=jnp.float32) * dv_ref[...]
    for i in range(2):
        o_ref[i] = u[:, 32 * i:32 * (i + 1)]


def _pre1_call(x, W1, dinv):
    return pl.pallas_call(
        _pre1_body,
        grid=(GRID,),
        in_specs=[
            pl.BlockSpec((BLK, 2), lambda i: (i, 0)),
            pl.BlockSpec((2, 64), lambda i: (0, 0)),
            pl.BlockSpec((BLK, 1), lambda i: (i, 0)),
        ],
        out_specs=pl.BlockSpec((2, BLK, 32), lambda i: (0, i, 0)),
        out_shape=jax.ShapeDtypeStruct((2, NN, 32), jnp.float32),
    )(x, W1, dinv)


def _make_post_pre(S_in, S_out, d_in, d_out):
    def body(a_ref, b_ref, dv_ref, w_ref, o_ref):
        acc = jnp.concatenate([a_ref[i] for i in range(S_in)], axis=1)
        dv = dv_ref[...]
        h = _gelu_f(dv * acc + b_ref[...])
        u = jnp.dot(h, w_ref[...], preferred_element_type=jnp.float32) * dv
        for i in range(S_out):
            o_ref[i] = u[:, 32 * i:32 * (i + 1)]

    def call(a, b, dinv, W):
        return pl.pallas_call(
            body,
            grid=(GRID,),
            in_specs=[
                pl.BlockSpec((S_in, BLK, 32), lambda i: (0, i, 0)),
                pl.BlockSpec((1, d_in), lambda i: (0, 0)),
                pl.BlockSpec((BLK, 1), lambda i: (i, 0)),
                pl.BlockSpec((d_in, d_out), lambda i: (0, 0)),
            ],
            out_specs=pl.BlockSpec((S_out, BLK, 32), lambda i: (0, i, 0)),
            out_shape=jax.ShapeDtypeStruct((S_out, NN, 32), jnp.float32),
        )(a, b.reshape(1, d_in), dinv, W)

    return call


def _make_bnstat(S_in, d):
    def body(a_ref, b_ref, dv_ref, z_ref, s_ref, q_ref):
        acc = jnp.concatenate([a_ref[i] for i in range(S_in)], axis=1)
        z = dv_ref[...] * acc + b_ref[...]
        z_ref[...] = z
        s_ref[0] = jnp.sum(z, axis=0, keepdims=True)
        q_ref[0] = jnp.sum(z * z, axis=0, keepdims=True)

    def call(a, b, dinv):
        return pl.pallas_call(
            body,
            grid=(GRID,),
            in_specs=[
                pl.BlockSpec((S_in, BLK, 32), lambda i: (0, i, 0)),
                pl.BlockSpec((1, d), lambda i: (0, 0)),
                pl.BlockSpec((BLK, 1), lambda i: (i, 0)),
            ],
            out_specs=[
                pl.BlockSpec((BLK, d), lambda i: (i, 0)),
                pl.BlockSpec((1, 1, d), lambda i: (i, 0, 0)),
                pl.BlockSpec((1, 1, d), lambda i: (i, 0, 0)),
            ],
            out_shape=[
                jax.ShapeDtypeStruct((NN, d), jnp.float32),
                jax.ShapeDtypeStruct((GRID, 1, d), jnp.float32),
                jax.ShapeDtypeStruct((GRID, 1, d), jnp.float32),
            ],
        )(a, b.reshape(1, d), dinv)

    return call


def _make_bnapply(d, S_out, d_out, with_pre):
    def body(z_ref, s_ref, q_ref, g_ref, be_ref, *rest):
        m = jnp.sum(s_ref[...], axis=0) * (1.0 / NN)
        v = jnp.sum(q_ref[...], axis=0) * (1.0 / NN) - m * m
        inv = lax.rsqrt(v + 1e-5)
        h = _gelu_f((z_ref[...] - m) * inv * g_ref[...] + be_ref[...])
        if with_pre:
            w_ref, dv_ref, o_ref = rest
            u = jnp.dot(h, w_ref[...],
                        preferred_element_type=jnp.float32) * dv_ref[...]
            for i in range(S_out):
                o_ref[i] = u[:, 32 * i:32 * (i + 1)]
        else:
            (o_ref,) = rest
            o_ref[...] = h

    def call(z, sums, sq, g, be, W=None, dinv=None):
        in_specs = [
            pl.BlockSpec((BLK, d), lambda i: (i, 0)),
            pl.BlockSpec((GRID, 1, d), lambda i: (0, 0, 0)),
            pl.BlockSpec((GRID, 1, d), lambda i: (0, 0, 0)),
            pl.BlockSpec((1, d), lambda i: (0, 0)),
            pl.BlockSpec((1, d), lambda i: (0, 0)),
        ]
        args = [z, sums, sq, g.reshape(1, d), be.reshape(1, d)]
        if with_pre:
            in_specs += [
                pl.BlockSpec((d, d_out), lambda i: (0, 0)),
                pl.BlockSpec((BLK, 1), lambda i: (i, 0)),
            ]
            args += [W, dinv]
            out_specs = pl.BlockSpec((S_out, BLK, 32), lambda i: (0, i, 0))
            out_shape = jax.ShapeDtypeStruct((S_out, NN, 32), jnp.float32)
        else:
            out_specs = pl.BlockSpec((BLK, d), lambda i: (i, 0))
            out_shape = jax.ShapeDtypeStruct((NN, d), jnp.float32)
        return pl.pallas_call(
            body,
            grid=(GRID,),
            in_specs=in_specs,
            out_specs=out_specs,
            out_shape=out_shape,
        )(*args)

    return call


def _head_body(p_ref, w1_ref, b1_ref, w2_ref, b2_ref, o_ref):
    pooled = jnp.max(p_ref[...], axis=0)
    t = jnp.dot(pooled, w1_ref[...],
                preferred_element_type=jnp.float32) + b1_ref[...]
    o_ref[...] = jnp.dot(t, w2_ref[...],
                         preferred_element_type=jnp.float32) + b2_ref[...]


def _head_call(parts, lin1_W, lin1_b, lin_W, lin_b):
    return pl.pallas_call(
        _head_body,
        out_shape=jax.ShapeDtypeStruct((NGROUPS, 2), jnp.float32),
    )(parts, lin1_W, lin1_b.reshape(1, 10), lin_W, lin_b.reshape(1, 2))


_bnstat64 = _make_bnstat(2, 64)
_bnstat128 = _make_bnstat(4, 128)
_bnapply64_pre = _make_bnapply(64, 2, 64, True)
_bnapply128_pre = _make_bnapply(128, 4, 128, True)
_bnapply64_out = _make_bnapply(64, 0, 0, False)
_post64_128 = _make_post_pre(2, 4, 64, 128)
_post128_64 = _make_post_pre(4, 2, 128, 64)


def kernel(x, ei, batch, W1, b1, W2, b2, W3, b3, W4, b4, W5, b5, g1, be1, g2,
           be2, g3, be3, lin1_W, lin1_b, lin_W, lin_b):
    f32 = jnp.float32
    src = ei[0]
    dst = ei[1]
    pad_idx = jnp.arange(EPAD - EE, dtype=jnp.int32) % 128
    src_p = jnp.concatenate([src, pad_idx]).reshape(EPAD // 128, 128)
    dst_p = jnp.concatenate([dst, pad_idx + NN]).reshape(EPAD // 128, 128)

    deg_parts = _deg_call(dst_p).reshape(2, NN, 16)
    dinv = _dinv_call(deg_parts)

    u1 = _pre1_call(x.astype(f32), W1, dinv)
    a1 = _spmm2(u1.reshape(2 * NN, 32), src_p, dst_p).reshape(2, NN, 32)
    z1, s1, q1 = _bnstat64(a1, b1, dinv)
    u2 = _bnapply64_pre(z1, s1, q1, g1, be1, W2, dinv)
    a2 = _spmm2(u2.reshape(2 * NN, 32), src_p, dst_p).reshape(2, NN, 32)
    u3 = _post64_128(a2, b2, dinv, W3)
    a3 = _spmm4(u3.reshape(4 * NN, 32), src_p, dst_p).reshape(4, NN, 32)
    z3, s3, q3 = _bnstat128(a3, b3, dinv)
    u4 = _bnapply128_pre(z3, s3, q3, g2, be2, W4, dinv)
    a4 = _spmm4(u4.reshape(4 * NN, 32), src_p, dst_p).reshape(4, NN, 32)
    u5 = _post128_64(a4, b4, dinv, W5)
    a5 = _spmm2(u5.reshape(2 * NN, 32), src_p, dst_p).reshape(2, NN, 32)
    z5, s5, q5 = _bnstat64(a5, b5, dinv)
    h5 = _bnapply64_out(z5, s5, q5, g3, be3)

    h5p = jnp.concatenate(
        [h5, jnp.full((NP2 - NN, 64), -jnp.inf, f32)], axis=0)
    bp = jnp.concatenate(
        [batch, jnp.full((NP2 - NN,), NGROUPS - 1, jnp.int32)])
    parts = _pool_call(h5p, bp)
    return _head_call(parts, lin1_W, lin1_b, lin_W, lin_b)


# speedup vs baseline: 17.2615x; 1.2952x over previous
import jax
import jax.numpy as jnp
from jax import lax
from jax.experimental import pallas as pl
from jax.experimental.pallas import tpu as pltpu
from jax.experimental.pallas import tpu_sc as plsc

NN = 50000
EE = 800000
EPAD = 819200
NPAD = NN + 128
NGROUPS = 64
BLK = 1000
GRID = NN // BLK
RPT = 3128
RPT_LAST = NN - 15 * RPT


def _row_split_copy(s, src_fn, dst_fn):
    @pl.when(s < 15)
    def _():
        pltpu.sync_copy(src_fn(RPT), dst_fn(RPT))

    @pl.when(s == 15)
    def _():
        pltpu.sync_copy(src_fn(RPT_LAST), dst_fn(RPT_LAST))


_SC_MESH = dict(core_axis_name="c", subcore_axis_name="s")


def _deg_body(dst2_hbm, out_hbm, ones_v, didx_v, acc_s):
    c = lax.axis_index("c")
    s = lax.axis_index("s")

    def fill(i, carry):
        ones_v[i, pl.ds(0, 16)] = jnp.full((16,), 1.0, jnp.float32)
        return carry

    lax.fori_loop(0, RPT, fill, 0, unroll=4)
    r0 = s * RPT
    _row_split_copy(s, lambda n: ones_v.at[pl.ds(0, n)],
                    lambda n: acc_s.at[pl.ds(r0, n)])
    plsc.subcore_barrier()

    def blk(b, carry):
        er0 = c * 3200 + s * 200 + b * 8
        pltpu.sync_copy(dst2_hbm.at[pl.ds(er0, 8)], didx_v)
        for j in range(8):
            pltpu.sync_copy(ones_v.at[pl.ds(0, 128)], acc_s.at[didx_v.at[j]],
                            add=True)
        return carry

    lax.fori_loop(0, 25, blk, 0)
    plsc.subcore_barrier()
    _row_split_copy(s, lambda n: acc_s.at[pl.ds(r0, n)],
                    lambda n: out_hbm.at[pl.ds(c * NN + r0, n)])


_deg_call = pl.kernel(
    _deg_body,
    out_type=jax.ShapeDtypeStruct((2 * NN, 16), jnp.float32),
    mesh=plsc.VectorSubcoreMesh(**_SC_MESH),
    compiler_params=pltpu.CompilerParams(use_tc_tiling_on_sc=False),
    scratch_types=[
        pltpu.VMEM((RPT, 16), jnp.float32),
        pltpu.VMEM((8, 128), jnp.int32),
        pltpu.VMEM_SHARED((NPAD, 16), jnp.float32),
    ],
)


def _make_spmm(S):
    S_pc = S // 2

    def body(u_hbm, srcoff_hbm, dst2_hbm, out_hbm, sidx_v, didx_v, rows_v,
             isem, sems, acc_s):
        c = lax.axis_index("c")
        s = lax.axis_index("s")

        def slice_body(sl_local, carry):
            sl = c * S_pc + sl_local
            base_row = sl * NN
            r0 = s * RPT
            _row_split_copy(s, lambda n: u_hbm.at[pl.ds(base_row + r0, n)],
                            lambda n: acc_s.at[pl.ds(r0, n)])
            plsc.subcore_barrier()

            srow0 = sl * 6400 + s * 400
            drow0 = s * 400
            pltpu.sync_copy(srcoff_hbm.at[pl.ds(srow0, 16)],
                            sidx_v.at[pl.ds(0, 16)])
            pltpu.sync_copy(dst2_hbm.at[pl.ds(drow0, 16)],
                            didx_v.at[pl.ds(0, 16)])

            def blk(b, bcarry):
                cur = lax.rem(b, 2) * 16
                nxt = lax.rem(b + 1, 2) * 16

                @pl.when(b > 0)
                def _():
                    pltpu.make_async_copy(
                        srcoff_hbm.at[pl.ds(srow0 + b * 16, 16)],
                        sidx_v.at[pl.ds(cur, 16)], isem).wait()
                    pltpu.make_async_copy(
                        dst2_hbm.at[pl.ds(drow0 + b * 16, 16)],
                        didx_v.at[pl.ds(cur, 16)], isem).wait()

                @pl.when(b < 24)
                def _():
                    pltpu.async_copy(
                        srcoff_hbm.at[pl.ds(srow0 + (b + 1) * 16, 16)],
                        sidx_v.at[pl.ds(nxt, 16)], isem)
                    pltpu.async_copy(
                        dst2_hbm.at[pl.ds(drow0 + (b + 1) * 16, 16)],
                        didx_v.at[pl.ds(nxt, 16)], isem)

                cps = [
                    pltpu.async_copy(u_hbm.at[sidx_v.at[cur + j]],
                                     rows_v.at[j], sems.at[j])
                    for j in range(4)
                ]
                for j in range(16):
                    jb = j % 4
                    cps[jb].wait()
                    pltpu.sync_copy(rows_v.at[jb],
                                    acc_s.at[didx_v.at[cur + j]], add=True)
                    if j + 4 < 16:
                        cps[jb] = pltpu.async_copy(
                            u_hbm.at[sidx_v.at[cur + j + 4]], rows_v.at[jb],
                            sems.at[jb])
                return bcarry

            lax.fori_loop(0, 25, blk, 0)
            plsc.subcore_barrier()
            _row_split_copy(s, lambda n: acc_s.at[pl.ds(r0, n)],
                            lambda n: out_hbm.at[pl.ds(base_row + r0, n)])
            plsc.subcore_barrier()
            return carry

        lax.fori_loop(0, S_pc, slice_body, 0)

    return pl.kernel(
        body,
        out_type=jax.ShapeDtypeStruct((S * NN, 32), jnp.float32),
        mesh=plsc.VectorSubcoreMesh(**_SC_MESH),
        compiler_params=pltpu.CompilerParams(use_tc_tiling_on_sc=False),
        scratch_types=[
            pltpu.VMEM((32, 128), jnp.int32),
            pltpu.VMEM((32, 128), jnp.int32),
            pltpu.VMEM((4, 128, 32), jnp.float32),
            pltpu.SemaphoreType.DMA,
            pltpu.SemaphoreType.DMA((4,)),
            pltpu.VMEM_SHARED((NPAD, 32), jnp.float32),
        ],
    )


_spmm2 = _make_spmm(2)
_spmm4 = _make_spmm(4)

NP2 = 50176
RPW = 1568
RBLK = 224
NBLKP = RPW // RBLK


def _pool_body(h_hbm, b_hbm, out_hbm, rows_v, bid_v, pacc_v):
    c = lax.axis_index("c")
    s = lax.axis_index("s")
    w = s * 2 + c

    def initp(i, carry):
        for j in range(4):
            pacc_v[i, pl.ds(j * 16, 16)] = jnp.full((16,), -jnp.inf,
                                                    jnp.float32)
        return carry

    lax.fori_loop(0, NGROUPS, initp, 0, unroll=4)
    pltpu.sync_copy(b_hbm.at[pl.ds(w * RPW, RPW)], bid_v.at[pl.ds(0, RPW)])

    def blk(bb, carry):
        pltpu.sync_copy(h_hbm.at[pl.ds(w * RPW + bb * RBLK, RBLK)], rows_v)

        def row(r, rcarry):
            gid = bid_v[pl.ds(bb * RBLK + r, 16)][0]
            for j in range(4):
                sl_ = pl.ds(j * 16, 16)
                pacc_v[gid, sl_] = jnp.maximum(pacc_v[gid, sl_],
                                               rows_v[r, sl_])
            return rcarry

        lax.fori_loop(0, RBLK, row, 0)
        return carry

    lax.fori_loop(0, NBLKP, blk, 0)
    pltpu.sync_copy(pacc_v, out_hbm.at[w])


_pool_call = pl.kernel(
    _pool_body,
    out_type=jax.ShapeDtypeStruct((32, NGROUPS, 64), jnp.float32),
    mesh=plsc.VectorSubcoreMesh(**_SC_MESH),
    compiler_params=pltpu.CompilerParams(use_tc_tiling_on_sc=False),
    scratch_types=[
        pltpu.VMEM((RBLK, 64), jnp.float32),
        pltpu.VMEM((RPW + 16,), jnp.int32),
        pltpu.VMEM((NGROUPS, 64), jnp.float32),
    ],
)


def _gelu_f(x):
    return x * 0.5 * (1.0 + lax.erf(x * 0.7071067811865476))


def _dinv_body(dp_ref, o_ref):
    d = dp_ref[0, :, 0:1] + dp_ref[1, :, 0:1] - 1.0
    o_ref[...] = lax.rsqrt(d)


def _dinv_call(deg_parts):
    return pl.pallas_call(
        _dinv_body,
        grid=(GRID,),
        in_specs=[pl.BlockSpec((2, BLK, 16), lambda i: (0, i, 0))],
        out_specs=pl.BlockSpec((BLK, 1), lambda i: (i, 0)),
        out_shape=jax.ShapeDtypeStruct((NN, 1), jnp.float32),
    )(deg_parts)


def _pre1_body(x_ref, w_ref, dv_ref, o_ref):
    u = jnp.dot(x_ref[...], w_ref[...],
                preferred_element_type=jnp.float32) * dv_ref[...]
    for i in range(2):
        o_ref[i] = u[:, 32 * i:32 * (i + 1)]


def _pre1_call(x, W1, dinv):
    return pl.pallas_call(
        _pre1_body,
        grid=(GRID,),
        in_specs=[
            pl.BlockSpec((BLK, 2), lambda i: (i, 0)),
            pl.BlockSpec((2, 64), lambda i: (0, 0)),
            pl.BlockSpec((BLK, 1), lambda i: (i, 0)),
        ],
        out_specs=pl.BlockSpec((2, BLK, 32), lambda i: (0, i, 0)),
        out_shape=jax.ShapeDtypeStruct((2, NN, 32), jnp.float32),
    )(x, W1, dinv)


def _make_post_pre(S_in, S_out, d_in, d_out):
    def body(a_ref, b_ref, dv_ref, w_ref, o_ref):
        acc = jnp.concatenate([a_ref[i] for i in range(S_in)], axis=1)
        dv = dv_ref[...]
        h = _gelu_f(dv * acc + b_ref[...])
        u = jnp.dot(h, w_ref[...], preferred_element_type=jnp.float32) * dv
        for i in range(S_out):
            o_ref[i] = u[:, 32 * i:32 * (i + 1)]

    def call(a, b, dinv, W):
        return pl.pallas_call(
            body,
            grid=(GRID,),
            in_specs=[
                pl.BlockSpec((S_in, BLK, 32), lambda i: (0, i, 0)),
                pl.BlockSpec((1, d_in), lambda i: (0, 0)),
                pl.BlockSpec((BLK, 1), lambda i: (i, 0)),
                pl.BlockSpec((d_in, d_out), lambda i: (0, 0)),
            ],
            out_specs=pl.BlockSpec((S_out, BLK, 32), lambda i: (0, i, 0)),
            out_shape=jax.ShapeDtypeStruct((S_out, NN, 32), jnp.float32),
        )(a, b.reshape(1, d_in), dinv, W)

    return call


def _make_bnstat(S_in, d):
    def body(a_ref, b_ref, dv_ref, z_ref, s_ref, q_ref):
        acc = jnp.concatenate([a_ref[i] for i in range(S_in)], axis=1)
        z = dv_ref[...] * acc + b_ref[...]
        z_ref[...] = z
        s_ref[0] = jnp.sum(z, axis=0, keepdims=True)
        q_ref[0] = jnp.sum(z * z, axis=0, keepdims=True)

    def call(a, b, dinv):
        return pl.pallas_call(
            body,
            grid=(GRID,),
            in_specs=[
                pl.BlockSpec((S_in, BLK, 32), lambda i: (0, i, 0)),
                pl.BlockSpec((1, d), lambda i: (0, 0)),
                pl.BlockSpec((BLK, 1), lambda i: (i, 0)),
            ],
            out_specs=[
                pl.BlockSpec((BLK, d), lambda i: (i, 0)),
                pl.BlockSpec((1, 1, d), lambda i: (i, 0, 0)),
                pl.BlockSpec((1, 1, d), lambda i: (i, 0, 0)),
            ],
            out_shape=[
                jax.ShapeDtypeStruct((NN, d), jnp.float32),
                jax.ShapeDtypeStruct((GRID, 1, d), jnp.float32),
                jax.ShapeDtypeStruct((GRID, 1, d), jnp.float32),
            ],
        )(a, b.reshape(1, d), dinv)

    return call


def _make_bnapply(d, S_out, d_out, with_pre):
    def body(z_ref, s_ref, q_ref, g_ref, be_ref, *rest):
        m = jnp.sum(s_ref[...], axis=0) * (1.0 / NN)
        v = jnp.sum(q_ref[...], axis=0) * (1.0 / NN) - m * m
        inv = lax.rsqrt(v + 1e-5)
        h = _gelu_f((z_ref[...] - m) * inv * g_ref[...] + be_ref[...])
        if with_pre:
            w_ref, dv_ref, o_ref = rest
            u = jnp.dot(h, w_ref[...],
                        preferred_element_type=jnp.float32) * dv_ref[...]
            for i in range(S_out):
                o_ref[i] = u[:, 32 * i:32 * (i + 1)]
        else:
            (o_ref,) = rest
            o_ref[...] = h

    def call(z, sums, sq, g, be, W=None, dinv=None):
        in_specs = [
            pl.BlockSpec((BLK, d), lambda i: (i, 0)),
            pl.BlockSpec((GRID, 1, d), lambda i: (0, 0, 0)),
            pl.BlockSpec((GRID, 1, d), lambda i: (0, 0, 0)),
            pl.BlockSpec((1, d), lambda i: (0, 0)),
            pl.BlockSpec((1, d), lambda i: (0, 0)),
        ]
        args = [z, sums, sq, g.reshape(1, d), be.reshape(1, d)]
        if with_pre:
            in_specs += [
                pl.BlockSpec((d, d_out), lambda i: (0, 0)),
                pl.BlockSpec((BLK, 1), lambda i: (i, 0)),
            ]
            args += [W, dinv]
            out_specs = pl.BlockSpec((S_out, BLK, 32), lambda i: (0, i, 0))
            out_shape = jax.ShapeDtypeStruct((S_out, NN, 32), jnp.float32)
        else:
            out_specs = pl.BlockSpec((BLK, d), lambda i: (i, 0))
            out_shape = jax.ShapeDtypeStruct((NN, d), jnp.float32)
        return pl.pallas_call(
            body,
            grid=(GRID,),
            in_specs=in_specs,
            out_specs=out_specs,
            out_shape=out_shape,
        )(*args)

    return call


def _head_body(p_ref, w1_ref, b1_ref, w2_ref, b2_ref, o_ref):
    pooled = jnp.max(p_ref[...], axis=0)
    t = jnp.dot(pooled, w1_ref[...],
                preferred_element_type=jnp.float32) + b1_ref[...]
    o_ref[...] = jnp.dot(t, w2_ref[...],
                         preferred_element_type=jnp.float32) + b2_ref[...]


def _head_call(parts, lin1_W, lin1_b, lin_W, lin_b):
    return pl.pallas_call(
        _head_body,
        out_shape=jax.ShapeDtypeStruct((NGROUPS, 2), jnp.float32),
    )(parts, lin1_W, lin1_b.reshape(1, 10), lin_W, lin_b.reshape(1, 2))


_bnstat64 = _make_bnstat(2, 64)
_bnstat128 = _make_bnstat(4, 128)
_bnapply64_pre = _make_bnapply(64, 2, 64, True)
_bnapply128_pre = _make_bnapply(128, 4, 128, True)
_bnapply64_out = _make_bnapply(64, 0, 0, False)
_post64_128 = _make_post_pre(2, 4, 64, 128)
_post128_64 = _make_post_pre(4, 2, 128, 64)


def kernel(x, ei, batch, W1, b1, W2, b2, W3, b3, W4, b4, W5, b5, g1, be1, g2,
           be2, g3, be3, lin1_W, lin1_b, lin_W, lin_b):
    f32 = jnp.float32
    src = ei[0]
    dst = ei[1]
    pad_idx = jnp.arange(EPAD - EE, dtype=jnp.int32) % 128
    src_p = jnp.concatenate([src, pad_idx]).reshape(EPAD // 128, 128)
    dst_p = jnp.concatenate([dst, pad_idx + NN]).reshape(EPAD // 128, 128)
    srcoff4 = jnp.concatenate(
        [src_p + k * NN for k in range(4)], axis=0)
    srcoff2 = srcoff4[:2 * (EPAD // 128)]

    deg_parts = _deg_call(dst_p).reshape(2, NN, 16)
    dinv = _dinv_call(deg_parts)

    u1 = _pre1_call(x.astype(f32), W1, dinv)
    a1 = _spmm2(u1.reshape(2 * NN, 32), srcoff2, dst_p).reshape(2, NN, 32)
    z1, s1, q1 = _bnstat64(a1, b1, dinv)
    u2 = _bnapply64_pre(z1, s1, q1, g1, be1, W2, dinv)
    a2 = _spmm2(u2.reshape(2 * NN, 32), srcoff2, dst_p).reshape(2, NN, 32)
    u3 = _post64_128(a2, b2, dinv, W3)
    a3 = _spmm4(u3.reshape(4 * NN, 32), srcoff4, dst_p).reshape(4, NN, 32)
    z3, s3, q3 = _bnstat128(a3, b3, dinv)
    u4 = _bnapply128_pre(z3, s3, q3, g2, be2, W4, dinv)
    a4 = _spmm4(u4.reshape(4 * NN, 32), srcoff4, dst_p).reshape(4, NN, 32)
    u5 = _post128_64(a4, b4, dinv, W5)
    a5 = _spmm2(u5.reshape(2 * NN, 32), srcoff2, dst_p).reshape(2, NN, 32)
    z5, s5, q5 = _bnstat64(a5, b5, dinv)
    h5 = _bnapply64_out(z5, s5, q5, g3, be3)

    h5p = jnp.concatenate(
        [h5, jnp.full((NP2 - NN, 64), -jnp.inf, f32)], axis=0)
    bp = jnp.concatenate(
        [batch, jnp.full((NP2 - NN,), NGROUPS - 1, jnp.int32)])
    parts = _pool_call(h5p, bp)
    return _head_call(parts, lin1_W, lin1_b, lin_W, lin_b)
